# baseline (speedup 1.0000x reference)
import functools

import jax
import jax.numpy as jnp
from jax import lax
from jax.experimental import pallas as pl
from jax.experimental.pallas import tpu as pltpu
from jax.experimental.pallas import tpu_sc as plsc

NC = 2
NS = 16
NW = NC * NS
BLK = 128
BM = 1024


def _pad_to(n, m):
    return (n + m - 1) // m * m


def _make_deg_kernel(npad, nblk):
    mesh = plsc.VectorSubcoreMesh(core_axis_name="c", subcore_axis_name="s", num_cores=NC, num_subcores=NS)
    rows_pt = npad // NS

    @functools.partial(
        pl.kernel,
        out_type=jax.ShapeDtypeStruct((NC, npad, 16), jnp.float32),
        mesh=mesh,
        scratch_types=[
            pltpu.VMEM_SHARED((npad, 16), jnp.float32),
            pltpu.VMEM((nblk, BLK), jnp.int32),
            pltpu.VMEM((BLK, 16), jnp.float32),
            pltpu.SemaphoreType.DMA,
        ],
    )
    def deg_kernel(dst_hbm, zeros_hbm, out_hbm, deg_sh, dstv, ones_v, ssem):
        c = lax.axis_index("c")
        s = lax.axis_index("s")
        wid = c * NS + s
        r0 = s * rows_pt
        pltpu.sync_copy(zeros_hbm.at[pl.ds(r0, rows_pt)],
                        deg_sh.at[pl.ds(r0, rows_pt)])
        pltpu.sync_copy(dst_hbm.at[wid], dstv)

        @pl.loop(0, BLK)
        def _(i):
            ones_v[i, :] = jnp.ones((16,), jnp.float32)

        plsc.subcore_barrier()

        @pl.loop(0, nblk, step=8)
        def _(g):
            for j in range(8):
                pltpu.async_copy(ones_v, deg_sh.at[dstv.at[g + j]],
                                 ssem, add=True)
            for j in range(8):
                pltpu.make_async_copy(ones_v, deg_sh.at[dstv.at[g + j]],
                                      ssem).wait()

        plsc.subcore_barrier()
        pltpu.sync_copy(deg_sh.at[pl.ds(r0, rows_pt)],
                        out_hbm.at[c, pl.ds(r0, rows_pt)])

    return deg_kernel


CH = 8


def _make_accum_kernel(npad, nblk, d):
    mesh = plsc.VectorSubcoreMesh(core_axis_name="c", subcore_axis_name="s", num_cores=NC, num_subcores=NS)
    rows_pt = npad // NS
    assert nblk % (2 * CH) == 0 and CH % 2 == 0

    @functools.partial(
        pl.kernel,
        out_type=jax.ShapeDtypeStruct((NC, npad, d), jnp.float32),
        mesh=mesh,
        scratch_types=[
            pltpu.VMEM_SHARED((npad, d), jnp.float32),
            pltpu.VMEM((2, CH, BLK), jnp.int32),
            pltpu.VMEM((2, CH, BLK), jnp.int32),
            pltpu.VMEM((2, BLK, d), jnp.float32),
            pltpu.SemaphoreType.DMA,
            pltpu.SemaphoreType.DMA,
            pltpu.SemaphoreType.DMA,
        ],
    )
    def accum_kernel(y_hbm, src_hbm, dst_hbm, zeros_hbm, out_hbm,
                     acc_sh, srcv, dstv, rows, isem, gsem0, gsem1):
        c = lax.axis_index("c")
        s = lax.axis_index("s")
        wid = c * NS + s
        r0 = s * rows_pt
        gsems = (gsem0, gsem1)
        pltpu.sync_copy(zeros_hbm.at[pl.ds(r0, rows_pt)],
                        acc_sh.at[pl.ds(r0, rows_pt)])
        pltpu.sync_copy(src_hbm.at[wid, pl.ds(0, CH)], srcv.at[0])
        pltpu.sync_copy(dst_hbm.at[wid, pl.ds(0, CH)], dstv.at[0])
        plsc.subcore_barrier()

        pltpu.async_copy(y_hbm.at[srcv.at[0, 0]], rows.at[0], gsem0)

        @pl.loop(0, nblk, step=2 * CH)
        def _(g):
            for half in range(2):
                nxt = g + half * CH + CH

                @pl.when(nxt < nblk)
                def _():
                    pltpu.async_copy(src_hbm.at[wid, pl.ds(nxt, CH)],
                                     srcv.at[1 - half], isem)
                    pltpu.async_copy(dst_hbm.at[wid, pl.ds(nxt, CH)],
                                     dstv.at[1 - half], isem)

                for j in range(CH):
                    rb = j % 2
                    pltpu.make_async_copy(y_hbm.at[srcv.at[half, j]],
                                          rows.at[rb], gsems[rb]).wait()
                    if j + 1 < CH:
                        pltpu.async_copy(y_hbm.at[srcv.at[half, j + 1]],
                                         rows.at[1 - rb], gsems[1 - rb])
                    else:
                        @pl.when(nxt < nblk)
                        def _():
                            pltpu.make_async_copy(
                                src_hbm.at[wid, pl.ds(nxt, CH)],
                                srcv.at[1 - half], isem).wait()
                            pltpu.make_async_copy(
                                dst_hbm.at[wid, pl.ds(nxt, CH)],
                                dstv.at[1 - half], isem).wait()
                            pltpu.async_copy(y_hbm.at[srcv.at[1 - half, 0]],
                                             rows.at[1 - rb], gsems[1 - rb])
                    pltpu.sync_copy(rows.at[rb],
                                    acc_sh.at[dstv.at[half, j]], add=True)

        plsc.subcore_barrier()
        pltpu.sync_copy(acc_sh.at[pl.ds(r0, rows_pt)],
                        out_hbm.at[c, pl.ds(r0, rows_pt)])

    return accum_kernel


def _matmul(x, w):
    m, k = x.shape
    _, n = w.shape

    def body(x_ref, w_ref, o_ref):
        o_ref[...] = jnp.dot(x_ref[...], w_ref[...],
                             preferred_element_type=jnp.float32)

    return pl.pallas_call(
        body,
        grid=(m // BM,),
        in_specs=[pl.BlockSpec((BM, k), lambda i: (i, 0)),
                  pl.BlockSpec((k, n), lambda i: (0, 0))],
        out_specs=pl.BlockSpec((BM, n), lambda i: (i, 0)),
        out_shape=jax.ShapeDtypeStruct((m, n), jnp.float32),
    )(x, w)


def _prescale(xw, d0, d1):
    m, n = xw.shape

    def body(xw_ref, d0_ref, d1_ref, o_ref):
        deg = d0_ref[:, :1] + d1_ref[:, :1] + 1.0
        o_ref[...] = lax.rsqrt(deg) * xw_ref[...]

    return pl.pallas_call(
        body,
        grid=(m // BM,),
        in_specs=[pl.BlockSpec((BM, n), lambda i: (i, 0)),
                  pl.BlockSpec((BM, 16), lambda i: (i, 0)),
                  pl.BlockSpec((BM, 16), lambda i: (i, 0))],
        out_specs=pl.BlockSpec((BM, n), lambda i: (i, 0)),
        out_shape=jax.ShapeDtypeStruct((m, n), jnp.float32),
    )(xw, d0, d1)


def _layer2(a0, a1, y1, d0, d1, b1, w2):
    m, n = y1.shape
    _, n2 = w2.shape

    def body(a0_ref, a1_ref, y1_ref, d0_ref, d1_ref, b1_ref, w2_ref, o_ref):
        deg = d0_ref[:, :1] + d1_ref[:, :1] + 1.0
        dis = lax.rsqrt(deg)
        h = dis * (a0_ref[...] + a1_ref[...] + y1_ref[...]) + b1_ref[...]
        h = jnp.maximum(h, 0.0)
        o_ref[...] = dis * jnp.dot(h, w2_ref[...],
                                   preferred_element_type=jnp.float32)

    return pl.pallas_call(
        body,
        grid=(m // BM,),
        in_specs=[pl.BlockSpec((BM, n), lambda i: (i, 0)),
                  pl.BlockSpec((BM, n), lambda i: (i, 0)),
                  pl.BlockSpec((BM, n), lambda i: (i, 0)),
                  pl.BlockSpec((BM, 16), lambda i: (i, 0)),
                  pl.BlockSpec((BM, 16), lambda i: (i, 0)),
                  pl.BlockSpec((1, n), lambda i: (0, 0)),
                  pl.BlockSpec((n, n2), lambda i: (0, 0))],
        out_specs=pl.BlockSpec((BM, n2), lambda i: (i, 0)),
        out_shape=jax.ShapeDtypeStruct((m, n2), jnp.float32),
    )(a0, a1, y1, d0, d1, b1, w2)


def _final(a0, a1, y2, d0, d1, b2):
    m, n = y2.shape

    def body(a0_ref, a1_ref, y2_ref, d0_ref, d1_ref, b2_ref, o_ref):
        deg = d0_ref[:, :1] + d1_ref[:, :1] + 1.0
        dis = lax.rsqrt(deg)
        o_ref[...] = dis * (a0_ref[...] + a1_ref[...] + y2_ref[...]) \
            + b2_ref[...]

    return pl.pallas_call(
        body,
        grid=(m // BM,),
        in_specs=[pl.BlockSpec((BM, n), lambda i: (i, 0)),
                  pl.BlockSpec((BM, n), lambda i: (i, 0)),
                  pl.BlockSpec((BM, n), lambda i: (i, 0)),
                  pl.BlockSpec((BM, 16), lambda i: (i, 0)),
                  pl.BlockSpec((BM, 16), lambda i: (i, 0)),
                  pl.BlockSpec((1, n), lambda i: (0, 0))],
        out_specs=pl.BlockSpec((BM, n), lambda i: (i, 0)),
        out_shape=jax.ShapeDtypeStruct((m, n), jnp.float32),
    )(a0, a1, y2, d0, d1, b2)


def kernel(x, W1, b1, W2, b2, edge_index):
    n, d_in = x.shape
    d_hid = W1.shape[1]
    d_out = W2.shape[1]
    e = edge_index.shape[1]

    assert BM % NS == 0
    npad = _pad_to(n + 1, BM)
    epad = _pad_to(e, NW * BLK * 2 * CH)
    nblk = epad // (NW * BLK)

    src = edge_index[0].astype(jnp.int32)
    dst = edge_index[1].astype(jnp.int32)
    if epad > e:
        pad = jnp.full((epad - e,), n, jnp.int32)
        src = jnp.concatenate([src, pad])
        dst = jnp.concatenate([dst, pad])
    src = src.reshape(NW, nblk, BLK)
    dst = dst.reshape(NW, nblk, BLK)

    d_msg = _pad_to(d_out, 128)
    W2p = jnp.concatenate(
        [W2, jnp.zeros((d_hid, d_msg - d_out), jnp.float32)], axis=1)
    b2p = jnp.concatenate([b2, jnp.zeros((d_msg - d_out,), jnp.float32)])

    x_pad = jnp.concatenate(
        [x, jnp.zeros((npad - n, d_in), jnp.float32)], axis=0)
    z16 = jnp.zeros((npad, 16), jnp.float32)
    zhid = jnp.zeros((npad, d_hid), jnp.float32)
    zout = jnp.zeros((npad, d_msg), jnp.float32)
    b1r = b1.reshape(1, d_hid)
    b2r = b2p.reshape(1, d_msg)

    dp = _make_deg_kernel(npad, nblk)(dst, z16)
    d0, d1 = dp[0], dp[1]

    xw1 = _matmul(x_pad, W1)
    y1 = _prescale(xw1, d0, d1)

    a1 = _make_accum_kernel(npad, nblk, d_hid)(y1, src, dst, zhid)
    y2 = _layer2(a1[0], a1[1], y1, d0, d1, b1r, W2p)

    a2 = _make_accum_kernel(npad, nblk, d_msg)(y2, src, dst, zout)
    out = _final(a2[0], a2[1], y2, d0, d1, b2r)
    return out[:n, :d_out]


# speedup vs baseline: 1.1664x; 1.1664x over previous
import functools

import jax
import jax.numpy as jnp
from jax import lax
from jax.experimental import pallas as pl
from jax.experimental.pallas import tpu as pltpu
from jax.experimental.pallas import tpu_sc as plsc

NC = 2
NS = 16
NW = NC * NS
BLK = 128
BM = 1024


def _pad_to(n, m):
    return (n + m - 1) // m * m


def _make_deg_kernel(npad, nblk):
    mesh = plsc.VectorSubcoreMesh(core_axis_name="c", subcore_axis_name="s", num_cores=NC, num_subcores=NS)
    rows_pt = npad // NS

    @functools.partial(
        pl.kernel,
        out_type=jax.ShapeDtypeStruct((NC, npad, 16), jnp.float32),
        mesh=mesh,
        scratch_types=[
            pltpu.VMEM_SHARED((npad, 16), jnp.float32),
            pltpu.VMEM((nblk, BLK), jnp.int32),
            pltpu.VMEM((BLK, 16), jnp.float32),
            pltpu.SemaphoreType.DMA,
        ],
    )
    def deg_kernel(dst_hbm, zeros_hbm, out_hbm, deg_sh, dstv, ones_v, ssem):
        c = lax.axis_index("c")
        s = lax.axis_index("s")
        wid = c * NS + s
        r0 = s * rows_pt
        pltpu.sync_copy(zeros_hbm.at[pl.ds(r0, rows_pt)],
                        deg_sh.at[pl.ds(r0, rows_pt)])
        pltpu.sync_copy(dst_hbm.at[wid], dstv)

        @pl.loop(0, BLK)
        def _(i):
            ones_v[i, :] = jnp.ones((16,), jnp.float32)

        plsc.subcore_barrier()

        @pl.loop(0, nblk, step=8)
        def _(g):
            for j in range(8):
                pltpu.async_copy(ones_v, deg_sh.at[dstv.at[g + j]],
                                 ssem, add=True)
            for j in range(8):
                pltpu.make_async_copy(ones_v, deg_sh.at[dstv.at[g + j]],
                                      ssem).wait()

        plsc.subcore_barrier()
        pltpu.sync_copy(deg_sh.at[pl.ds(r0, rows_pt)],
                        out_hbm.at[c, pl.ds(r0, rows_pt)])

    return deg_kernel


CH = 8


def _make_accum_kernel(npad, nblk, d):
    mesh = plsc.VectorSubcoreMesh(core_axis_name="c", subcore_axis_name="s", num_cores=NC, num_subcores=NS)
    rows_pt = npad // NS
    assert nblk % (2 * CH) == 0 and CH % 2 == 0

    @functools.partial(
        pl.kernel,
        out_type=jax.ShapeDtypeStruct((NC, npad, d), jnp.float32),
        mesh=mesh,
        scratch_types=[
            pltpu.VMEM_SHARED((npad, d), jnp.float32),
            pltpu.VMEM((2, CH, BLK), jnp.int32),
            pltpu.VMEM((2, CH, BLK), jnp.int32),
            pltpu.VMEM((2, BLK, d), jnp.float32),
            pltpu.SemaphoreType.DMA,
            pltpu.SemaphoreType.DMA,
            pltpu.SemaphoreType.DMA,
        ],
    )
    def accum_kernel(y_hbm, src_hbm, dst_hbm, zeros_hbm, out_hbm,
                     acc_sh, srcv, dstv, rows, isem, gsem0, gsem1):
        c = lax.axis_index("c")
        s = lax.axis_index("s")
        y_c = y_hbm.at[c]
        wid = c * NS + s
        r0 = s * rows_pt
        gsems = (gsem0, gsem1)
        pltpu.sync_copy(zeros_hbm.at[pl.ds(r0, rows_pt)],
                        acc_sh.at[pl.ds(r0, rows_pt)])
        pltpu.sync_copy(src_hbm.at[wid, pl.ds(0, CH)], srcv.at[0])
        pltpu.sync_copy(dst_hbm.at[wid, pl.ds(0, CH)], dstv.at[0])
        plsc.subcore_barrier()

        pltpu.async_copy(y_c.at[srcv.at[0, 0]], rows.at[0], gsem0)

        @pl.loop(0, nblk, step=2 * CH)
        def _(g):
            for half in range(2):
                nxt = g + half * CH + CH

                @pl.when(nxt < nblk)
                def _():
                    pltpu.async_copy(src_hbm.at[wid, pl.ds(nxt, CH)],
                                     srcv.at[1 - half], isem)
                    pltpu.async_copy(dst_hbm.at[wid, pl.ds(nxt, CH)],
                                     dstv.at[1 - half], isem)

                for j in range(CH):
                    rb = j % 2
                    pltpu.make_async_copy(y_c.at[srcv.at[half, j]],
                                          rows.at[rb], gsems[rb]).wait()
                    if j + 1 < CH:
                        pltpu.async_copy(y_c.at[srcv.at[half, j + 1]],
                                         rows.at[1 - rb], gsems[1 - rb])
                    else:
                        @pl.when(nxt < nblk)
                        def _():
                            pltpu.make_async_copy(
                                src_hbm.at[wid, pl.ds(nxt, CH)],
                                srcv.at[1 - half], isem).wait()
                            pltpu.make_async_copy(
                                dst_hbm.at[wid, pl.ds(nxt, CH)],
                                dstv.at[1 - half], isem).wait()
                            pltpu.async_copy(y_c.at[srcv.at[1 - half, 0]],
                                             rows.at[1 - rb], gsems[1 - rb])
                    pltpu.sync_copy(rows.at[rb],
                                    acc_sh.at[dstv.at[half, j]], add=True)

        plsc.subcore_barrier()
        pltpu.sync_copy(acc_sh.at[pl.ds(r0, rows_pt)],
                        out_hbm.at[c, pl.ds(r0, rows_pt)])

    return accum_kernel


def _matmul(x, w):
    m, k = x.shape
    _, n = w.shape

    def body(x_ref, w_ref, o_ref):
        o_ref[...] = jnp.dot(x_ref[...], w_ref[...],
                             preferred_element_type=jnp.float32)

    return pl.pallas_call(
        body,
        grid=(m // BM,),
        in_specs=[pl.BlockSpec((BM, k), lambda i: (i, 0)),
                  pl.BlockSpec((k, n), lambda i: (0, 0))],
        out_specs=pl.BlockSpec((BM, n), lambda i: (i, 0)),
        out_shape=jax.ShapeDtypeStruct((m, n), jnp.float32),
    )(x, w)


def _prescale(xw, d0, d1):
    m, n = xw.shape

    def body(xw_ref, d0_ref, d1_ref, o_ref):
        deg = d0_ref[:, :1] + d1_ref[:, :1] + 1.0
        o_ref[...] = lax.rsqrt(deg) * xw_ref[...]

    return pl.pallas_call(
        body,
        grid=(m // BM,),
        in_specs=[pl.BlockSpec((BM, n), lambda i: (i, 0)),
                  pl.BlockSpec((BM, 16), lambda i: (i, 0)),
                  pl.BlockSpec((BM, 16), lambda i: (i, 0))],
        out_specs=pl.BlockSpec((BM, n), lambda i: (i, 0)),
        out_shape=jax.ShapeDtypeStruct((m, n), jnp.float32),
    )(xw, d0, d1)


def _layer2(a0, a1, y1, d0, d1, b1, w2):
    m, n = y1.shape
    _, n2 = w2.shape

    def body(a0_ref, a1_ref, y1_ref, d0_ref, d1_ref, b1_ref, w2_ref, o_ref):
        deg = d0_ref[:, :1] + d1_ref[:, :1] + 1.0
        dis = lax.rsqrt(deg)
        h = dis * (a0_ref[...] + a1_ref[...] + y1_ref[...]) + b1_ref[...]
        h = jnp.maximum(h, 0.0)
        o_ref[...] = dis * jnp.dot(h, w2_ref[...],
                                   preferred_element_type=jnp.float32)

    return pl.pallas_call(
        body,
        grid=(m // BM,),
        in_specs=[pl.BlockSpec((BM, n), lambda i: (i, 0)),
                  pl.BlockSpec((BM, n), lambda i: (i, 0)),
                  pl.BlockSpec((BM, n), lambda i: (i, 0)),
                  pl.BlockSpec((BM, 16), lambda i: (i, 0)),
                  pl.BlockSpec((BM, 16), lambda i: (i, 0)),
                  pl.BlockSpec((1, n), lambda i: (0, 0)),
                  pl.BlockSpec((n, n2), lambda i: (0, 0))],
        out_specs=pl.BlockSpec((BM, n2), lambda i: (i, 0)),
        out_shape=jax.ShapeDtypeStruct((m, n2), jnp.float32),
    )(a0, a1, y1, d0, d1, b1, w2)


def _final(a0, a1, y2, d0, d1, b2):
    m, n = y2.shape

    def body(a0_ref, a1_ref, y2_ref, d0_ref, d1_ref, b2_ref, o_ref):
        deg = d0_ref[:, :1] + d1_ref[:, :1] + 1.0
        dis = lax.rsqrt(deg)
        o_ref[...] = dis * (a0_ref[...] + a1_ref[...] + y2_ref[...]) \
            + b2_ref[...]

    return pl.pallas_call(
        body,
        grid=(m // BM,),
        in_specs=[pl.BlockSpec((BM, n), lambda i: (i, 0)),
                  pl.BlockSpec((BM, n), lambda i: (i, 0)),
                  pl.BlockSpec((BM, n), lambda i: (i, 0)),
                  pl.BlockSpec((BM, 16), lambda i: (i, 0)),
                  pl.BlockSpec((BM, 16), lambda i: (i, 0)),
                  pl.BlockSpec((1, n), lambda i: (0, 0))],
        out_specs=pl.BlockSpec((BM, n), lambda i: (i, 0)),
        out_shape=jax.ShapeDtypeStruct((m, n), jnp.float32),
    )(a0, a1, y2, d0, d1, b2)


def kernel(x, W1, b1, W2, b2, edge_index):
    n, d_in = x.shape
    d_hid = W1.shape[1]
    d_out = W2.shape[1]
    e = edge_index.shape[1]

    assert BM % NS == 0
    npad = _pad_to(n + 1, BM)
    epad = _pad_to(e, NW * BLK * 2 * CH)
    nblk = epad // (NW * BLK)

    src = edge_index[0].astype(jnp.int32)
    dst = edge_index[1].astype(jnp.int32)
    if epad > e:
        pad = jnp.full((epad - e,), n, jnp.int32)
        src = jnp.concatenate([src, pad])
        dst = jnp.concatenate([dst, pad])
    src = src.reshape(NW, nblk, BLK)
    dst = dst.reshape(NW, nblk, BLK)

    d_msg = _pad_to(d_out, 128)
    W2p = jnp.concatenate(
        [W2, jnp.zeros((d_hid, d_msg - d_out), jnp.float32)], axis=1)
    b2p = jnp.concatenate([b2, jnp.zeros((d_msg - d_out,), jnp.float32)])

    x_pad = jnp.concatenate(
        [x, jnp.zeros((npad - n, d_in), jnp.float32)], axis=0)
    z16 = jnp.zeros((npad, 16), jnp.float32)
    zhid = jnp.zeros((npad, d_hid), jnp.float32)
    zout = jnp.zeros((npad, d_msg), jnp.float32)
    b1r = b1.reshape(1, d_hid)
    b2r = b2p.reshape(1, d_msg)

    dp = _make_deg_kernel(npad, nblk)(dst, z16)
    d0, d1 = dp[0], dp[1]

    xw1 = _matmul(x_pad, W1)
    y1 = _prescale(xw1, d0, d1)

    a1 = _make_accum_kernel(npad, nblk, d_hid)(jnp.stack([y1, y1]), src, dst, zhid)
    y2 = _layer2(a1[0], a1[1], y1, d0, d1, b1r, W2p)

    a2 = _make_accum_kernel(npad, nblk, d_msg)(jnp.stack([y2, y2]), src, dst, zout)
    out = _final(a2[0], a2[1], y2, d0, d1, b2r)
    return out[:n, :d_out]


# speedup vs baseline: 2.6437x; 2.2666x over previous
import functools

import jax
import jax.numpy as jnp
from jax import lax
from jax.experimental import pallas as pl
from jax.experimental.pallas import tpu as pltpu
from jax.experimental.pallas import tpu_sc as plsc

NC = 2
NS = 16
NW = NC * NS
BLK = 128
BM = 1024


def _pad_to(n, m):
    return (n + m - 1) // m * m


def _make_deg_kernel(npad, nblk):
    mesh = plsc.VectorSubcoreMesh(core_axis_name="c", subcore_axis_name="s", num_cores=NC, num_subcores=NS)
    rows_pt = npad // NS

    @functools.partial(
        pl.kernel,
        out_type=jax.ShapeDtypeStruct((NC, npad, 16), jnp.float32),
        mesh=mesh,
        scratch_types=[
            pltpu.VMEM_SHARED((npad, 16), jnp.float32),
            pltpu.VMEM((nblk, BLK), jnp.int32),
            pltpu.VMEM((BLK, 16), jnp.float32),
            pltpu.SemaphoreType.DMA,
        ],
    )
    def deg_kernel(dst_hbm, zeros_hbm, out_hbm, deg_sh, dstv, ones_v, ssem):
        c = lax.axis_index("c")
        s = lax.axis_index("s")
        wid = c * NS + s
        r0 = s * rows_pt
        pltpu.sync_copy(zeros_hbm.at[pl.ds(r0, rows_pt)],
                        deg_sh.at[pl.ds(r0, rows_pt)])
        pltpu.sync_copy(dst_hbm.at[wid], dstv)

        @pl.loop(0, BLK)
        def _(i):
            ones_v[i, :] = jnp.ones((16,), jnp.float32)

        plsc.subcore_barrier()

        @pl.loop(0, nblk, step=8)
        def _(g):
            for j in range(8):
                pltpu.async_copy(ones_v, deg_sh.at[dstv.at[g + j]],
                                 ssem, add=True)
            for j in range(8):
                pltpu.make_async_copy(ones_v, deg_sh.at[dstv.at[g + j]],
                                      ssem).wait()

        plsc.subcore_barrier()
        pltpu.sync_copy(deg_sh.at[pl.ds(r0, rows_pt)],
                        out_hbm.at[c, pl.ds(r0, rows_pt)])

    return deg_kernel


CH = 8


def _make_accum_kernel(npad, nblk, dh):
    mesh = plsc.VectorSubcoreMesh(core_axis_name="c", subcore_axis_name="s", num_cores=NC, num_subcores=NS)
    rows_pt = npad // NS
    assert nblk % (2 * CH) == 0 and CH % 2 == 0

    @functools.partial(
        pl.kernel,
        out_type=jax.ShapeDtypeStruct((NC, npad, dh), jnp.float32),
        mesh=mesh,
        scratch_types=[
            pltpu.VMEM_SHARED((npad, dh), jnp.float32),
            pltpu.VMEM_SHARED((npad, dh), jnp.float32),
            pltpu.VMEM((2, CH, BLK), jnp.int32),
            pltpu.VMEM((2, CH, BLK), jnp.int32),
            pltpu.VMEM((2, BLK, dh), jnp.float32),
            pltpu.SemaphoreType.DMA,
            pltpu.SemaphoreType.DMA,
            pltpu.SemaphoreType.DMA,
        ],
    )
    def accum_kernel(y_hbm, src_hbm, dst_hbm, zeros_hbm, out_hbm,
                     y_sh, acc_sh, srcv, dstv, rows, isem, gsem0, gsem1):
        c = lax.axis_index("c")
        s = lax.axis_index("s")
        r0 = s * rows_pt
        gsems = (gsem0, gsem1)
        pltpu.sync_copy(y_hbm.at[c, pl.ds(r0, rows_pt)],
                        y_sh.at[pl.ds(r0, rows_pt)])
        pltpu.sync_copy(zeros_hbm.at[pl.ds(r0, rows_pt)],
                        acc_sh.at[pl.ds(r0, rows_pt)])
        pltpu.sync_copy(src_hbm.at[s, pl.ds(0, CH)], srcv.at[0])
        pltpu.sync_copy(dst_hbm.at[s, pl.ds(0, CH)], dstv.at[0])
        plsc.subcore_barrier()

        pltpu.async_copy(y_sh.at[srcv.at[0, 0]], rows.at[0], gsem0)

        @pl.loop(0, nblk, step=2 * CH)
        def _(g):
            for half in range(2):
                nxt = g + half * CH + CH

                @pl.when(nxt < nblk)
                def _():
                    pltpu.async_copy(src_hbm.at[s, pl.ds(nxt, CH)],
                                     srcv.at[1 - half], isem)
                    pltpu.async_copy(dst_hbm.at[s, pl.ds(nxt, CH)],
                                     dstv.at[1 - half], isem)

                for j in range(CH):
                    rb = j % 2
                    pltpu.make_async_copy(y_sh.at[srcv.at[half, j]],
                                          rows.at[rb], gsems[rb]).wait()
                    if j + 1 < CH:
                        pltpu.async_copy(y_sh.at[srcv.at[half, j + 1]],
                                         rows.at[1 - rb], gsems[1 - rb])
                    else:
                        @pl.when(nxt < nblk)
                        def _():
                            pltpu.make_async_copy(
                                src_hbm.at[s, pl.ds(nxt, CH)],
                                srcv.at[1 - half], isem).wait()
                            pltpu.make_async_copy(
                                dst_hbm.at[s, pl.ds(nxt, CH)],
                                dstv.at[1 - half], isem).wait()
                            pltpu.async_copy(y_sh.at[srcv.at[1 - half, 0]],
                                             rows.at[1 - rb], gsems[1 - rb])
                    pltpu.sync_copy(rows.at[rb],
                                    acc_sh.at[dstv.at[half, j]], add=True)

        plsc.subcore_barrier()
        pltpu.sync_copy(acc_sh.at[pl.ds(r0, rows_pt)],
                        out_hbm.at[c, pl.ds(r0, rows_pt)])

    return accum_kernel


def _matmul(x, w):
    m, k = x.shape
    _, n = w.shape

    def body(x_ref, w_ref, o_ref):
        o_ref[...] = jnp.dot(x_ref[...], w_ref[...],
                             preferred_element_type=jnp.float32)

    return pl.pallas_call(
        body,
        grid=(m // BM,),
        in_specs=[pl.BlockSpec((BM, k), lambda i: (i, 0)),
                  pl.BlockSpec((k, n), lambda i: (0, 0))],
        out_specs=pl.BlockSpec((BM, n), lambda i: (i, 0)),
        out_shape=jax.ShapeDtypeStruct((m, n), jnp.float32),
    )(x, w)


def _prescale(xw, d0, d1):
    m, n = xw.shape

    def body(xw_ref, d0_ref, d1_ref, o_ref):
        deg = d0_ref[:, :1] + d1_ref[:, :1] + 1.0
        o_ref[...] = lax.rsqrt(deg) * xw_ref[...]

    return pl.pallas_call(
        body,
        grid=(m // BM,),
        in_specs=[pl.BlockSpec((BM, n), lambda i: (i, 0)),
                  pl.BlockSpec((BM, 16), lambda i: (i, 0)),
                  pl.BlockSpec((BM, 16), lambda i: (i, 0))],
        out_specs=pl.BlockSpec((BM, n), lambda i: (i, 0)),
        out_shape=jax.ShapeDtypeStruct((m, n), jnp.float32),
    )(xw, d0, d1)


def _layer2(a0, a1, y1, d0, d1, b1, w2):
    m, n = y1.shape
    nh = n // 2
    _, n2 = w2.shape

    def body(a0_ref, a1_ref, y1_ref, d0_ref, d1_ref, b1_ref, w2_ref, o_ref):
        deg = d0_ref[:, :1] + d1_ref[:, :1] + 1.0
        dis = lax.rsqrt(deg)
        acc = jnp.concatenate([a0_ref[...], a1_ref[...]], axis=1)
        h = dis * (acc + y1_ref[...]) + b1_ref[...]
        h = jnp.maximum(h, 0.0)
        o_ref[...] = dis * jnp.dot(h, w2_ref[...],
                                   preferred_element_type=jnp.float32)

    return pl.pallas_call(
        body,
        grid=(m // BM,),
        in_specs=[pl.BlockSpec((BM, nh), lambda i: (i, 0)),
                  pl.BlockSpec((BM, nh), lambda i: (i, 0)),
                  pl.BlockSpec((BM, n), lambda i: (i, 0)),
                  pl.BlockSpec((BM, 16), lambda i: (i, 0)),
                  pl.BlockSpec((BM, 16), lambda i: (i, 0)),
                  pl.BlockSpec((1, n), lambda i: (0, 0)),
                  pl.BlockSpec((n, n2), lambda i: (0, 0))],
        out_specs=pl.BlockSpec((BM, n2), lambda i: (i, 0)),
        out_shape=jax.ShapeDtypeStruct((m, n2), jnp.float32),
    )(a0, a1, y1, d0, d1, b1, w2)


def _final(a0, a1, y2, d0, d1, b2):
    m, n = y2.shape
    nh = n // 2

    def body(a0_ref, a1_ref, y2_ref, d0_ref, d1_ref, b2_ref, o_ref):
        deg = d0_ref[:, :1] + d1_ref[:, :1] + 1.0
        dis = lax.rsqrt(deg)
        acc = jnp.concatenate([a0_ref[...], a1_ref[...]], axis=1)
        o_ref[...] = dis * (acc + y2_ref[...]) + b2_ref[...]

    return pl.pallas_call(
        body,
        grid=(m // BM,),
        in_specs=[pl.BlockSpec((BM, nh), lambda i: (i, 0)),
                  pl.BlockSpec((BM, nh), lambda i: (i, 0)),
                  pl.BlockSpec((BM, n), lambda i: (i, 0)),
                  pl.BlockSpec((BM, 16), lambda i: (i, 0)),
                  pl.BlockSpec((BM, 16), lambda i: (i, 0)),
                  pl.BlockSpec((1, n), lambda i: (0, 0))],
        out_specs=pl.BlockSpec((BM, n), lambda i: (i, 0)),
        out_shape=jax.ShapeDtypeStruct((m, n), jnp.float32),
    )(a0, a1, y2, d0, d1, b2)


def kernel(x, W1, b1, W2, b2, edge_index):
    n, d_in = x.shape
    d_hid = W1.shape[1]
    d_out = W2.shape[1]
    e = edge_index.shape[1]

    assert BM % NS == 0
    npad = _pad_to(n + 1, BM)
    epad = _pad_to(e, NW * BLK * 2 * CH)
    nblk_deg = epad // (NW * BLK)
    nblk = epad // (NS * BLK)

    src = edge_index[0].astype(jnp.int32)
    dst = edge_index[1].astype(jnp.int32)
    if epad > e:
        pad = jnp.full((epad - e,), n, jnp.int32)
        src = jnp.concatenate([src, pad])
        dst = jnp.concatenate([dst, pad])
    dst_d = dst.reshape(NW, nblk_deg, BLK)
    src = src.reshape(NS, nblk, BLK)
    dst = dst.reshape(NS, nblk, BLK)

    dh1 = d_hid // NC
    dh2 = d_out // NC

    x_pad = jnp.concatenate(
        [x, jnp.zeros((npad - n, d_in), jnp.float32)], axis=0)
    z16 = jnp.zeros((npad, 16), jnp.float32)
    zh1 = jnp.zeros((npad, dh1), jnp.float32)
    zh2 = jnp.zeros((npad, dh2), jnp.float32)
    b1r = b1.reshape(1, d_hid)
    b2r = b2.reshape(1, d_out)

    dp = _make_deg_kernel(npad, nblk_deg)(dst_d, z16)
    d0, d1 = dp[0], dp[1]

    xw1 = _matmul(x_pad, W1)
    y1 = _prescale(xw1, d0, d1)
    y1s = y1.reshape(npad, NC, dh1).transpose(1, 0, 2)

    a1 = _make_accum_kernel(npad, nblk, dh1)(y1s, src, dst, zh1)
    y2 = _layer2(a1[0], a1[1], y1, d0, d1, b1r, W2)
    y2s = y2.reshape(npad, NC, dh2).transpose(1, 0, 2)

    a2 = _make_accum_kernel(npad, nblk, dh2)(y2s, src, dst, zh2)
    out = _final(a2[0], a2[1], y2, d0, d1, b2r)
    return out[:n]
